# baseline (speedup 1.0000x reference)
import dataclasses
import functools

import jax
import jax.numpy as jnp
from jax import lax
from jax.experimental import pallas as pl
from jax.experimental.pallas import tpu as pltpu
from jax.experimental.pallas import tpu_sc as plsc

R = 16384
C = 1000
B = 200
NC, NS, L = 2, 16, 16
NW = NC * NS
ROWS_PER_W = R // NW
BLK = 32
NBLK = ROWS_PER_W // BLK
ELEMS = BLK * B
CHUNKS = ELEMS // L


def kernel(input1, input2):
    idx = input2.astype(jnp.int32).reshape(-1)
    tbl = input1.reshape(-1)
    rowbase = (jnp.arange(ELEMS, dtype=jnp.int32) // B) * C

    mesh = plsc.VectorSubcoreMesh(core_axis_name="c", subcore_axis_name="s")
    cp = pltpu.CompilerParams()
    if "needs_layout_passes" in pltpu.CompilerParams.__dataclass_fields__:
        cp = dataclasses.replace(cp, needs_layout_passes=False)

    @functools.partial(
        pl.kernel,
        compiler_params=cp,
        out_type=jax.ShapeDtypeStruct((R * B,), jnp.float32),
        mesh=mesh,
        scratch_types=[
            pltpu.VMEM((BLK * C,), jnp.float32),
            pltpu.VMEM((ELEMS,), jnp.int32),
            pltpu.VMEM((ELEMS,), jnp.float32),
            pltpu.VMEM((ELEMS,), jnp.int32),
        ],
    )
    def k(tbl_hbm, idx_hbm, rb_hbm, out_hbm, rows_v, idx_v, out_v, rb_v):
        wid = lax.axis_index("s") * NC + lax.axis_index("c")
        pltpu.sync_copy(rb_hbm, rb_v)

        @pl.loop(0, NBLK)
        def _(g):
            blk0 = (wid * ROWS_PER_W + g * BLK)
            pltpu.sync_copy(tbl_hbm.at[pl.ds(blk0 * C, BLK * C)], rows_v)
            pltpu.sync_copy(idx_hbm.at[pl.ds(blk0 * B, ELEMS)], idx_v)

            @pl.loop(0, CHUNKS)
            def _(c):
                s = pl.ds(c * L, L)
                flat = idx_v[s] + rb_v[s]
                out_v[s] = plsc.load_gather(rows_v, [flat])

            pltpu.sync_copy(out_v, out_hbm.at[pl.ds(blk0 * B, ELEMS)])

    out = k(tbl, idx, rowbase)
    return out.reshape(R, B)


# speedup vs baseline: 1.4679x; 1.4679x over previous
import dataclasses
import functools

import jax
import jax.numpy as jnp
from jax import lax
from jax.experimental import pallas as pl
from jax.experimental.pallas import tpu as pltpu
from jax.experimental.pallas import tpu_sc as plsc

R = 16384
C = 1000
B = 200
NC, NS, L = 2, 16, 16
NW = NC * NS
ROWS_PER_W = R // NW
BLK = 32
NBLK = ROWS_PER_W // BLK
FULL = B // L
TAIL = B - L


def kernel(input1, input2):
    idx = input2.astype(jnp.int32)

    mesh = plsc.VectorSubcoreMesh(core_axis_name="c", subcore_axis_name="s")
    cp = pltpu.CompilerParams()
    if "needs_layout_passes" in pltpu.CompilerParams.__dataclass_fields__:
        cp = dataclasses.replace(cp, needs_layout_passes=False)

    @functools.partial(
        pl.kernel,
        compiler_params=cp,
        out_type=jax.ShapeDtypeStruct((R, B), jnp.float32),
        mesh=mesh,
        scratch_types=[
            pltpu.VMEM((BLK, C), jnp.float32),
            pltpu.VMEM((BLK, B), jnp.int32),
            pltpu.VMEM((BLK, B), jnp.float32),
        ],
    )
    def k(tbl_hbm, idx_hbm, out_hbm, rows_v, idx_v, out_v):
        wid = lax.axis_index("s") * NC + lax.axis_index("c")

        @pl.loop(0, NBLK)
        def _(g):
            blk0 = wid * ROWS_PER_W + g * BLK
            pltpu.sync_copy(tbl_hbm.at[pl.ds(blk0, BLK)], rows_v)
            pltpu.sync_copy(idx_hbm.at[pl.ds(blk0, BLK)], idx_v)

            @pl.loop(0, BLK)
            def _(r):
                rsplat = jnp.full((L,), r, jnp.int32)

                @pl.loop(0, FULL)
                def _(c):
                    s = pl.ds(c * L, L)
                    col = idx_v[r, s]
                    out_v[r, s] = plsc.load_gather(rows_v, [rsplat, col])

                st = pl.ds(TAIL, L)
                colt = idx_v[r, st]
                out_v[r, st] = plsc.load_gather(rows_v, [rsplat, colt])

            pltpu.sync_copy(out_v, out_hbm.at[pl.ds(blk0, BLK)])

    return k(input1, idx)
